# baseline (speedup 1.0000x reference)
import functools

import jax
import jax.numpy as jnp
from jax import lax
from jax.experimental import pallas as pl
from jax.experimental.pallas import tpu as pltpu
from jax.experimental.pallas import tpu_sc as plsc

HIDDEN = 768
NPAIR = HIDDEN // 32
EPS = 1e-12
GCHUNK = 32
SPLITS = (8, 16, 20, 20)


def _sc_gather_body(ntok, ids_hbm, tok_hbm, out_hbm, idx_v, rows_v, rows_b,
                    gsem, osem):
    info = plsc.get_sparse_core_info()
    nc = info.num_cores
    wid = lax.axis_index("s") * nc + lax.axis_index("c")
    nw = nc * info.num_subcores
    per_w = ntok // nw
    nchunk = per_w // GCHUNK
    base = wid * per_w
    pltpu.sync_copy(ids_hbm.at[pl.ds(base, per_w)], idx_v)

    def fire_gather(c, buf):
        return pltpu.async_copy(
            tok_hbm.at[idx_v.at[pl.ds(c * GCHUNK, GCHUNK)]], rows_v.at[buf],
            gsem)

    def fire_out(c, buf):
        return pltpu.async_copy(
            rows_b.at[buf], out_hbm.at[pl.ds(base + c * GCHUNK, GCHUNK)],
            osem)

    def drain_g(buf):
        pltpu.make_async_copy(
            tok_hbm.at[idx_v.at[pl.ds(0, GCHUNK)]], rows_v.at[buf],
            gsem).wait()

    def drain_o(buf):
        pltpu.make_async_copy(rows_b.at[buf],
                              out_hbm.at[pl.ds(base, GCHUNK)], osem).wait()

    def convert(buf):
        half = jnp.int32(0x8000)
        himask = jnp.int32(-65536)

        def row_body(t, _):
            @plsc.parallel_loop(0, HIDDEN // 32, unroll=8)
            def _(j):
                a = rows_v[buf, t, pl.ds(j * 16, 16)]
                b = rows_v[buf, t, pl.ds(HIDDEN // 2 + j * 16, 16)]
                ai = lax.bitcast_convert_type(a, jnp.int32) + half
                bi = lax.bitcast_convert_type(b, jnp.int32) + half
                w = lax.shift_right_logical(ai, 16) | (bi & himask)
                rows_b[buf, t, pl.ds(j * 16, 16)] = w

            return 0

        lax.fori_loop(0, GCHUNK, row_body, 0)

    fire_gather(0, 0)
    fire_gather(1, 1)
    pairs = nchunk // 2

    def pair_body(i, _):
        c0 = 2 * i
        for buf in (0, 1):
            @pl.when(i > 0)
            def _():
                drain_o(buf)

            drain_g(buf)
            convert(buf)
            fire_out(c0 + buf, buf)

            @pl.when(i < pairs - 1)
            def _():
                fire_gather(c0 + 2 + buf, buf)

        return 0

    lax.fori_loop(0, pairs, pair_body, 0)
    drain_o(0)
    drain_o(1)


def _ln_math(gath_ref, tids_ref, pos_ref, type_ref, gamma_ref, beta_ref,
             o_ref):
    H2 = HIDDEN // 2
    w = gath_ref[0]
    xa = lax.bitcast_convert_type(lax.shift_left(w, 16), jnp.float32)
    xb = lax.bitcast_convert_type(w & jnp.int32(-65536), jnp.float32)
    tf = tids_ref[0].astype(jnp.float32).T
    pos = pos_ref[...]
    t0 = type_ref[0, :][None, :]
    dlt = (type_ref[1, :] - type_ref[0, :])[None, :]
    xa = xa + pos[:, :H2] + t0[:, :H2] + tf * dlt[:, :H2]
    xb = xb + pos[:, H2:] + t0[:, H2:] + tf * dlt[:, H2:]
    inv_n = jnp.float32(1.0 / HIDDEN)
    mean = (jnp.sum(xa, axis=-1, keepdims=True)
            + jnp.sum(xb, axis=-1, keepdims=True)) * inv_n
    xa = xa - mean
    xb = xb - mean
    var = (jnp.sum(xa * xa, axis=-1, keepdims=True)
           + jnp.sum(xb * xb, axis=-1, keepdims=True)) * inv_n
    rstd = lax.rsqrt(var + EPS)
    gamma = gamma_ref[...][None, :]
    beta = beta_ref[...][None, :]
    o_ref[0, :, :H2] = (xa * rstd) * gamma[:, :H2] + beta[:, :H2]
    o_ref[0, :, H2:] = (xb * rstd) * gamma[:, H2:] + beta[:, H2:]


def _tc_ln_chain(prev_ref, gath_ref, tids_ref, pos_ref, type_ref, gamma_ref,
                 beta_ref, o_ref):
    del prev_ref
    _ln_math(gath_ref, tids_ref, pos_ref, type_ref, gamma_ref, beta_ref,
             o_ref)


def kernel(input_ids, token_type_ids, token_table, pos_table, type_table,
           gamma, beta):
    bsz, seq = input_ids.shape
    ntok = bsz * seq
    ids = input_ids.reshape(-1).astype(jnp.int32)
    tids = token_type_ids.reshape(bsz, 1, seq).astype(jnp.int32)

    mesh = plsc.VectorSubcoreMesh(core_axis_name="c", subcore_axis_name="s")
    nw = mesh.num_cores * mesh.num_subcores

    def make_gather(tq):
        return pl.kernel(
            functools.partial(_sc_gather_body, tq),
            out_type=jax.ShapeDtypeStruct((tq, HIDDEN // 2), jnp.int32),
            mesh=mesh,
            scratch_types=[
                pltpu.VMEM((tq // nw,), jnp.int32),
                pltpu.VMEM((2, GCHUNK, HIDDEN), jnp.float32),
                pltpu.VMEM((2, GCHUNK, HIDDEN // 2), jnp.int32),
                pltpu.SemaphoreType.DMA,
                pltpu.SemaphoreType.DMA,
            ],
        )

    gather_calls = {}
    gathered = []
    t0_off = 0
    for nb in SPLITS:
        tq = nb * seq
        if tq not in gather_calls:
            gather_calls[tq] = make_gather(tq)
        gathered.append(gather_calls[tq](ids[t0_off:t0_off + tq],
                                         token_table))
        t0_off += tq

    common_specs = [
        pl.BlockSpec((1, 1, seq), lambda i: (i, 0, 0)),
        pl.BlockSpec((seq, HIDDEN), lambda i: (0, 0)),
        pl.BlockSpec((2, HIDDEN), lambda i: (0, 0)),
        pl.BlockSpec((HIDDEN,), lambda i: (0,)),
        pl.BlockSpec((HIDDEN,), lambda i: (0,)),
    ]
    out_shape = jax.ShapeDtypeStruct((bsz, seq, HIDDEN), jnp.float32)

    out = None
    b_off = 0
    for k, nb in enumerate(SPLITS):
        gk = gathered[k].reshape(nb, seq, HIDDEN // 2)
        tk = tids[b_off:b_off + nb]
        om = functools.partial(lambda bo, i: (bo + i, 0, 0), b_off)
        if out is None:
            out = pl.pallas_call(
                _ln_math,
                grid=(nb,),
                in_specs=[pl.BlockSpec((1, seq, HIDDEN // 2),
                                       lambda i: (i, 0, 0))] + common_specs,
                out_specs=pl.BlockSpec((1, seq, HIDDEN), om),
                out_shape=out_shape,
            )(gk, tk, pos_table, type_table, gamma, beta)
        else:
            out = pl.pallas_call(
                _tc_ln_chain,
                grid=(nb,),
                in_specs=[
                    pl.BlockSpec(memory_space=pl.ANY),
                    pl.BlockSpec((1, seq, HIDDEN // 2), lambda i: (i, 0, 0)),
                ] + common_specs,
                out_specs=pl.BlockSpec((1, seq, HIDDEN), om),
                out_shape=out_shape,
                input_output_aliases={0: 0},
            )(out, gk, tk, pos_table, type_table, gamma, beta)
        b_off += nb
    return out


# speedup vs baseline: 1.0382x; 1.0382x over previous
import functools

import jax
import jax.numpy as jnp
from jax import lax
from jax.experimental import pallas as pl
from jax.experimental.pallas import tpu as pltpu
from jax.experimental.pallas import tpu_sc as plsc

HIDDEN = 768
NPAIR = HIDDEN // 32
EPS = 1e-12
GCHUNK = 32
SPLITS = (16, 16, 16, 16)
TCB = 2


def _sc_gather_body(toff, ntok, ids_hbm, tok_hbm, out_hbm, idx_v, rows_v, rows_b,
                    gsem, osem):
    info = plsc.get_sparse_core_info()
    nc = info.num_cores
    wid = lax.axis_index("s") * nc + lax.axis_index("c")
    nw = nc * info.num_subcores
    per_w = ntok // nw
    nchunk = per_w // GCHUNK
    base = wid * per_w
    pltpu.sync_copy(ids_hbm.at[pl.ds(toff + base, per_w)], idx_v)

    def fire_gather(c, buf):
        return pltpu.async_copy(
            tok_hbm.at[idx_v.at[pl.ds(c * GCHUNK, GCHUNK)]], rows_v.at[buf],
            gsem)

    def fire_out(c, buf):
        return pltpu.async_copy(
            rows_b.at[buf], out_hbm.at[pl.ds(base + c * GCHUNK, GCHUNK)],
            osem)

    def drain_g(buf):
        pltpu.make_async_copy(
            tok_hbm.at[idx_v.at[pl.ds(0, GCHUNK)]], rows_v.at[buf],
            gsem).wait()

    def drain_o(buf):
        pltpu.make_async_copy(rows_b.at[buf],
                              out_hbm.at[pl.ds(base, GCHUNK)], osem).wait()

    def convert(buf):
        half = jnp.int32(0x8000)
        himask = jnp.int32(-65536)

        def row_body(t, _):
            @plsc.parallel_loop(0, HIDDEN // 32, unroll=8)
            def _(j):
                a = rows_v[buf, t, pl.ds(j * 16, 16)]
                b = rows_v[buf, t, pl.ds(HIDDEN // 2 + j * 16, 16)]
                ai = lax.bitcast_convert_type(a, jnp.int32) + half
                bi = lax.bitcast_convert_type(b, jnp.int32) + half
                w = lax.shift_right_logical(ai, 16) | (bi & himask)
                rows_b[buf, t, pl.ds(j * 16, 16)] = w

            return 0

        lax.fori_loop(0, GCHUNK, row_body, 0)

    fire_gather(0, 0)
    fire_gather(1, 1)
    pairs = nchunk // 2

    def pair_body(i, _):
        c0 = 2 * i
        for buf in (0, 1):
            @pl.when(i > 0)
            def _():
                drain_o(buf)

            drain_g(buf)
            convert(buf)
            fire_out(c0 + buf, buf)

            @pl.when(i < pairs - 1)
            def _():
                fire_gather(c0 + 2 + buf, buf)

        return 0

    lax.fori_loop(0, pairs, pair_body, 0)
    drain_o(0)
    drain_o(1)


def _ln_math(gath_ref, tids_ref, pos_ref, type_ref, gamma_ref, beta_ref,
             o_ref):
    H2 = HIDDEN // 2
    for r in range(TCB):
        _ln_row(gath_ref, tids_ref, pos_ref, type_ref, gamma_ref, beta_ref,
                o_ref, r)


def _ln_row(gath_ref, tids_ref, pos_ref, type_ref, gamma_ref, beta_ref,
            o_ref, r):
    H2 = HIDDEN // 2
    w = gath_ref[r]
    xa = lax.bitcast_convert_type(lax.shift_left(w, 16), jnp.float32)
    xb = lax.bitcast_convert_type(w & jnp.int32(-65536), jnp.float32)
    tf = tids_ref[r].astype(jnp.float32).T
    pos = pos_ref[...]
    t0 = type_ref[0, :][None, :]
    dlt = (type_ref[1, :] - type_ref[0, :])[None, :]
    xa = xa + pos[:, :H2] + t0[:, :H2] + tf * dlt[:, :H2]
    xb = xb + pos[:, H2:] + t0[:, H2:] + tf * dlt[:, H2:]
    inv_n = jnp.float32(1.0 / HIDDEN)
    mean = (jnp.sum(xa, axis=-1, keepdims=True)
            + jnp.sum(xb, axis=-1, keepdims=True)) * inv_n
    xa = xa - mean
    xb = xb - mean
    var = (jnp.sum(xa * xa, axis=-1, keepdims=True)
           + jnp.sum(xb * xb, axis=-1, keepdims=True)) * inv_n
    rstd = lax.rsqrt(var + EPS)
    gamma = gamma_ref[...][None, :]
    beta = beta_ref[...][None, :]
    o_ref[r, :, :H2] = (xa * rstd) * gamma[:, :H2] + beta[:, :H2]
    o_ref[r, :, H2:] = (xb * rstd) * gamma[:, H2:] + beta[:, H2:]


def _tc_ln_chain(prev_ref, gath_ref, tids_ref, pos_ref, type_ref, gamma_ref,
                 beta_ref, o_ref):
    del prev_ref
    _ln_math(gath_ref, tids_ref, pos_ref, type_ref, gamma_ref, beta_ref,
             o_ref)


def kernel(input_ids, token_type_ids, token_table, pos_table, type_table,
           gamma, beta):
    bsz, seq = input_ids.shape
    ntok = bsz * seq
    ids = input_ids.reshape(-1).astype(jnp.int32)
    tids = token_type_ids.reshape(bsz, 1, seq).astype(jnp.int32)

    mesh = plsc.VectorSubcoreMesh(core_axis_name="c", subcore_axis_name="s")
    nw = mesh.num_cores * mesh.num_subcores

    def make_gather(toff, tq):
        return pl.kernel(
            functools.partial(_sc_gather_body, toff, tq),
            out_type=jax.ShapeDtypeStruct((tq, HIDDEN // 2), jnp.int32),
            mesh=mesh,
            scratch_types=[
                pltpu.VMEM((tq // nw,), jnp.int32),
                pltpu.VMEM((2, GCHUNK, HIDDEN), jnp.float32),
                pltpu.VMEM((2, GCHUNK, HIDDEN // 2), jnp.int32),
                pltpu.SemaphoreType.DMA,
                pltpu.SemaphoreType.DMA,
            ],
        )

    gathered = []
    t0_off = 0
    for nb in SPLITS:
        tq = nb * seq
        gathered.append(make_gather(t0_off, tq)(ids, token_table))
        t0_off += tq

    common_specs = [
        pl.BlockSpec((TCB, 1, seq), lambda i: (i, 0, 0)),
        pl.BlockSpec((seq, HIDDEN), lambda i: (0, 0)),
        pl.BlockSpec((2, HIDDEN), lambda i: (0, 0)),
        pl.BlockSpec((HIDDEN,), lambda i: (0,)),
        pl.BlockSpec((HIDDEN,), lambda i: (0,)),
    ]
    out_shape = jax.ShapeDtypeStruct((bsz, seq, HIDDEN), jnp.float32)

    out = None
    b_off = 0
    for k, nb in enumerate(SPLITS):
        gk = gathered[k].reshape(nb, seq, HIDDEN // 2)
        tk = tids[b_off:b_off + nb]
        om = functools.partial(lambda bo, i: (bo + i, 0, 0), b_off // TCB)
        if out is None:
            out = pl.pallas_call(
                _ln_math,
                grid=(nb // TCB,),
                in_specs=[pl.BlockSpec((TCB, seq, HIDDEN // 2),
                                       lambda i: (i, 0, 0))] + common_specs,
                out_specs=pl.BlockSpec((TCB, seq, HIDDEN), om),
                out_shape=out_shape,
            )(gk, tk, pos_table, type_table, gamma, beta)
        else:
            out = pl.pallas_call(
                _tc_ln_chain,
                grid=(nb // TCB,),
                in_specs=[
                    pl.BlockSpec(memory_space=pl.ANY),
                    pl.BlockSpec((TCB, seq, HIDDEN // 2),
                                 lambda i: (i, 0, 0)),
                ] + common_specs,
                out_specs=pl.BlockSpec((TCB, seq, HIDDEN), om),
                out_shape=out_shape,
                input_output_aliases={0: 0},
            )(out, gk, tk, pos_table, type_table, gamma, beta)
        b_off += nb
    return out


# speedup vs baseline: 1.0666x; 1.0274x over previous
import functools

import jax
import jax.numpy as jnp
from jax import lax
from jax.experimental import pallas as pl
from jax.experimental.pallas import tpu as pltpu
from jax.experimental.pallas import tpu_sc as plsc

HIDDEN = 768
NPAIR = HIDDEN // 32
EPS = 1e-12
GCHUNK = 32
SPLITS = (16, 16, 16, 16)
TCB = 4


def _sc_gather_body(toff, ntok, ids_hbm, tok_hbm, out_hbm, idx_v, rows_v, rows_b,
                    gsem, osem):
    info = plsc.get_sparse_core_info()
    nc = info.num_cores
    wid = lax.axis_index("s") * nc + lax.axis_index("c")
    nw = nc * info.num_subcores
    per_w = ntok // nw
    nchunk = per_w // GCHUNK
    base = wid * per_w
    pltpu.sync_copy(ids_hbm.at[pl.ds(toff + base, per_w)], idx_v)

    def fire_gather(c, buf):
        return pltpu.async_copy(
            tok_hbm.at[idx_v.at[pl.ds(c * GCHUNK, GCHUNK)]], rows_v.at[buf],
            gsem)

    def fire_out(c, buf):
        return pltpu.async_copy(
            rows_b.at[buf], out_hbm.at[pl.ds(base + c * GCHUNK, GCHUNK)],
            osem)

    def drain_g(buf):
        pltpu.make_async_copy(
            tok_hbm.at[idx_v.at[pl.ds(0, GCHUNK)]], rows_v.at[buf],
            gsem).wait()

    def drain_o(buf):
        pltpu.make_async_copy(rows_b.at[buf],
                              out_hbm.at[pl.ds(base, GCHUNK)], osem).wait()

    def convert(buf):
        half = jnp.int32(0x8000)
        himask = jnp.int32(-65536)

        def row_body(t, _):
            @plsc.parallel_loop(0, HIDDEN // 32, unroll=8)
            def _(j):
                a = rows_v[buf, t, pl.ds(j * 16, 16)]
                b = rows_v[buf, t, pl.ds(HIDDEN // 2 + j * 16, 16)]
                ai = lax.bitcast_convert_type(a, jnp.int32) + half
                bi = lax.bitcast_convert_type(b, jnp.int32) + half
                w = lax.shift_right_logical(ai, 16) | (bi & himask)
                rows_b[buf, t, pl.ds(j * 16, 16)] = w

            return 0

        lax.fori_loop(0, GCHUNK, row_body, 0)

    fire_gather(0, 0)
    fire_gather(1, 1)
    pairs = nchunk // 2

    def pair_body(i, _):
        c0 = 2 * i
        for buf in (0, 1):
            @pl.when(i > 0)
            def _():
                drain_o(buf)

            drain_g(buf)
            convert(buf)
            fire_out(c0 + buf, buf)

            @pl.when(i < pairs - 1)
            def _():
                fire_gather(c0 + 2 + buf, buf)

        return 0

    lax.fori_loop(0, pairs, pair_body, 0)
    drain_o(0)
    drain_o(1)


def _ln_math(gath_ref, tids_ref, pos_ref, type_ref, gamma_ref, beta_ref,
             o_ref):
    H2 = HIDDEN // 2
    for r in range(TCB):
        _ln_row(gath_ref, tids_ref, pos_ref, type_ref, gamma_ref, beta_ref,
                o_ref, r)


def _ln_row(gath_ref, tids_ref, pos_ref, type_ref, gamma_ref, beta_ref,
            o_ref, r):
    H2 = HIDDEN // 2
    w = gath_ref[r]
    xa = lax.bitcast_convert_type(lax.shift_left(w, 16), jnp.float32)
    xb = lax.bitcast_convert_type(w & jnp.int32(-65536), jnp.float32)
    tf = tids_ref[r].astype(jnp.float32).T
    pos = pos_ref[...]
    t0 = type_ref[0, :][None, :]
    dlt = (type_ref[1, :] - type_ref[0, :])[None, :]
    xa = xa + pos[:, :H2] + t0[:, :H2] + tf * dlt[:, :H2]
    xb = xb + pos[:, H2:] + t0[:, H2:] + tf * dlt[:, H2:]
    inv_n = jnp.float32(1.0 / HIDDEN)
    mean = (jnp.sum(xa, axis=-1, keepdims=True)
            + jnp.sum(xb, axis=-1, keepdims=True)) * inv_n
    xa = xa - mean
    xb = xb - mean
    var = (jnp.sum(xa * xa, axis=-1, keepdims=True)
           + jnp.sum(xb * xb, axis=-1, keepdims=True)) * inv_n
    rstd = lax.rsqrt(var + EPS)
    gamma = gamma_ref[...][None, :]
    beta = beta_ref[...][None, :]
    o_ref[r, :, :H2] = (xa * rstd) * gamma[:, :H2] + beta[:, :H2]
    o_ref[r, :, H2:] = (xb * rstd) * gamma[:, H2:] + beta[:, H2:]


def _tc_ln_chain(prev_ref, gath_ref, tids_ref, pos_ref, type_ref, gamma_ref,
                 beta_ref, o_ref):
    del prev_ref
    _ln_math(gath_ref, tids_ref, pos_ref, type_ref, gamma_ref, beta_ref,
             o_ref)


def kernel(input_ids, token_type_ids, token_table, pos_table, type_table,
           gamma, beta):
    bsz, seq = input_ids.shape
    ntok = bsz * seq
    ids = input_ids.reshape(-1).astype(jnp.int32)
    tids = token_type_ids.reshape(bsz, 1, seq).astype(jnp.int32)

    mesh = plsc.VectorSubcoreMesh(core_axis_name="c", subcore_axis_name="s")
    nw = mesh.num_cores * mesh.num_subcores

    def make_gather(toff, tq):
        return pl.kernel(
            functools.partial(_sc_gather_body, toff, tq),
            out_type=jax.ShapeDtypeStruct((tq, HIDDEN // 2), jnp.int32),
            mesh=mesh,
            scratch_types=[
                pltpu.VMEM((tq // nw,), jnp.int32),
                pltpu.VMEM((2, GCHUNK, HIDDEN), jnp.float32),
                pltpu.VMEM((2, GCHUNK, HIDDEN // 2), jnp.int32),
                pltpu.SemaphoreType.DMA,
                pltpu.SemaphoreType.DMA,
            ],
        )

    gathered = []
    t0_off = 0
    for nb in SPLITS:
        tq = nb * seq
        gathered.append(make_gather(t0_off, tq)(ids, token_table))
        t0_off += tq

    common_specs = [
        pl.BlockSpec((TCB, 1, seq), lambda i: (i, 0, 0)),
        pl.BlockSpec((seq, HIDDEN), lambda i: (0, 0)),
        pl.BlockSpec((2, HIDDEN), lambda i: (0, 0)),
        pl.BlockSpec((HIDDEN,), lambda i: (0,)),
        pl.BlockSpec((HIDDEN,), lambda i: (0,)),
    ]
    out_shape = jax.ShapeDtypeStruct((bsz, seq, HIDDEN), jnp.float32)

    out = None
    b_off = 0
    for k, nb in enumerate(SPLITS):
        gk = gathered[k].reshape(nb, seq, HIDDEN // 2)
        tk = tids[b_off:b_off + nb]
        om = functools.partial(lambda bo, i: (bo + i, 0, 0), b_off // TCB)
        if out is None:
            out = pl.pallas_call(
                _ln_math,
                grid=(nb // TCB,),
                in_specs=[pl.BlockSpec((TCB, seq, HIDDEN // 2),
                                       lambda i: (i, 0, 0))] + common_specs,
                out_specs=pl.BlockSpec((TCB, seq, HIDDEN), om),
                out_shape=out_shape,
            )(gk, tk, pos_table, type_table, gamma, beta)
        else:
            out = pl.pallas_call(
                _tc_ln_chain,
                grid=(nb // TCB,),
                in_specs=[
                    pl.BlockSpec(memory_space=pl.ANY),
                    pl.BlockSpec((TCB, seq, HIDDEN // 2),
                                 lambda i: (i, 0, 0)),
                ] + common_specs,
                out_specs=pl.BlockSpec((TCB, seq, HIDDEN), om),
                out_shape=out_shape,
                input_output_aliases={0: 0},
            )(out, gk, tk, pos_table, type_table, gamma, beta)
        b_off += nb
    return out


# speedup vs baseline: 1.0959x; 1.0274x over previous
import functools

import jax
import jax.numpy as jnp
from jax import lax
from jax.experimental import pallas as pl
from jax.experimental.pallas import tpu as pltpu
from jax.experimental.pallas import tpu_sc as plsc

HIDDEN = 768
NPAIR = HIDDEN // 32
EPS = 1e-12
GCHUNK = 32
SPLITS = (16, 16, 16, 16)
TCB = 8


def _sc_gather_body(toff, ntok, ids_hbm, tok_hbm, out_hbm, idx_v, rows_v, rows_b,
                    gsem, osem):
    info = plsc.get_sparse_core_info()
    nc = info.num_cores
    wid = lax.axis_index("s") * nc + lax.axis_index("c")
    nw = nc * info.num_subcores
    per_w = ntok // nw
    nchunk = per_w // GCHUNK
    base = wid * per_w
    pltpu.sync_copy(ids_hbm.at[pl.ds(toff + base, per_w)], idx_v)

    def fire_gather(c, buf):
        return pltpu.async_copy(
            tok_hbm.at[idx_v.at[pl.ds(c * GCHUNK, GCHUNK)]], rows_v.at[buf],
            gsem)

    def fire_out(c, buf):
        return pltpu.async_copy(
            rows_b.at[buf], out_hbm.at[pl.ds(base + c * GCHUNK, GCHUNK)],
            osem)

    def drain_g(buf):
        pltpu.make_async_copy(
            tok_hbm.at[idx_v.at[pl.ds(0, GCHUNK)]], rows_v.at[buf],
            gsem).wait()

    def drain_o(buf):
        pltpu.make_async_copy(rows_b.at[buf],
                              out_hbm.at[pl.ds(base, GCHUNK)], osem).wait()

    def convert(buf):
        half = jnp.int32(0x8000)
        himask = jnp.int32(-65536)

        def row_body(t, _):
            @plsc.parallel_loop(0, HIDDEN // 32, unroll=8)
            def _(j):
                a = rows_v[buf, t, pl.ds(j * 16, 16)]
                b = rows_v[buf, t, pl.ds(HIDDEN // 2 + j * 16, 16)]
                ai = lax.bitcast_convert_type(a, jnp.int32) + half
                bi = lax.bitcast_convert_type(b, jnp.int32) + half
                w = lax.shift_right_logical(ai, 16) | (bi & himask)
                rows_b[buf, t, pl.ds(j * 16, 16)] = w

            return 0

        lax.fori_loop(0, GCHUNK, row_body, 0)

    fire_gather(0, 0)
    fire_gather(1, 1)
    pairs = nchunk // 2

    def pair_body(i, _):
        c0 = 2 * i
        for buf in (0, 1):
            @pl.when(i > 0)
            def _():
                drain_o(buf)

            drain_g(buf)
            convert(buf)
            fire_out(c0 + buf, buf)

            @pl.when(i < pairs - 1)
            def _():
                fire_gather(c0 + 2 + buf, buf)

        return 0

    lax.fori_loop(0, pairs, pair_body, 0)
    drain_o(0)
    drain_o(1)


def _ln_math(gath_ref, tids_ref, pos_ref, type_ref, gamma_ref, beta_ref,
             o_ref):
    H2 = HIDDEN // 2
    for r in range(TCB):
        _ln_row(gath_ref, tids_ref, pos_ref, type_ref, gamma_ref, beta_ref,
                o_ref, r)


def _ln_row(gath_ref, tids_ref, pos_ref, type_ref, gamma_ref, beta_ref,
            o_ref, r):
    H2 = HIDDEN // 2
    w = gath_ref[r]
    xa = lax.bitcast_convert_type(lax.shift_left(w, 16), jnp.float32)
    xb = lax.bitcast_convert_type(w & jnp.int32(-65536), jnp.float32)
    tf = tids_ref[r].astype(jnp.float32).T
    pos = pos_ref[...]
    t0 = type_ref[0, :][None, :]
    dlt = (type_ref[1, :] - type_ref[0, :])[None, :]
    xa = xa + pos[:, :H2] + t0[:, :H2] + tf * dlt[:, :H2]
    xb = xb + pos[:, H2:] + t0[:, H2:] + tf * dlt[:, H2:]
    inv_n = jnp.float32(1.0 / HIDDEN)
    mean = (jnp.sum(xa, axis=-1, keepdims=True)
            + jnp.sum(xb, axis=-1, keepdims=True)) * inv_n
    xa = xa - mean
    xb = xb - mean
    var = (jnp.sum(xa * xa, axis=-1, keepdims=True)
           + jnp.sum(xb * xb, axis=-1, keepdims=True)) * inv_n
    rstd = lax.rsqrt(var + EPS)
    gamma = gamma_ref[...][None, :]
    beta = beta_ref[...][None, :]
    o_ref[r, :, :H2] = (xa * rstd) * gamma[:, :H2] + beta[:, :H2]
    o_ref[r, :, H2:] = (xb * rstd) * gamma[:, H2:] + beta[:, H2:]


def _tc_ln_chain(prev_ref, gath_ref, tids_ref, pos_ref, type_ref, gamma_ref,
                 beta_ref, o_ref):
    del prev_ref
    _ln_math(gath_ref, tids_ref, pos_ref, type_ref, gamma_ref, beta_ref,
             o_ref)


def kernel(input_ids, token_type_ids, token_table, pos_table, type_table,
           gamma, beta):
    bsz, seq = input_ids.shape
    ntok = bsz * seq
    ids = input_ids.reshape(-1).astype(jnp.int32)
    tids = token_type_ids.reshape(bsz, 1, seq).astype(jnp.int32)

    mesh = plsc.VectorSubcoreMesh(core_axis_name="c", subcore_axis_name="s")
    nw = mesh.num_cores * mesh.num_subcores

    def make_gather(toff, tq):
        return pl.kernel(
            functools.partial(_sc_gather_body, toff, tq),
            out_type=jax.ShapeDtypeStruct((tq, HIDDEN // 2), jnp.int32),
            mesh=mesh,
            scratch_types=[
                pltpu.VMEM((tq // nw,), jnp.int32),
                pltpu.VMEM((2, GCHUNK, HIDDEN), jnp.float32),
                pltpu.VMEM((2, GCHUNK, HIDDEN // 2), jnp.int32),
                pltpu.SemaphoreType.DMA,
                pltpu.SemaphoreType.DMA,
            ],
        )

    gathered = []
    t0_off = 0
    for nb in SPLITS:
        tq = nb * seq
        gathered.append(make_gather(t0_off, tq)(ids, token_table))
        t0_off += tq

    common_specs = [
        pl.BlockSpec((TCB, 1, seq), lambda i: (i, 0, 0)),
        pl.BlockSpec((seq, HIDDEN), lambda i: (0, 0)),
        pl.BlockSpec((2, HIDDEN), lambda i: (0, 0)),
        pl.BlockSpec((HIDDEN,), lambda i: (0,)),
        pl.BlockSpec((HIDDEN,), lambda i: (0,)),
    ]
    out_shape = jax.ShapeDtypeStruct((bsz, seq, HIDDEN), jnp.float32)

    out = None
    b_off = 0
    for k, nb in enumerate(SPLITS):
        gk = gathered[k].reshape(nb, seq, HIDDEN // 2)
        tk = tids[b_off:b_off + nb]
        om = functools.partial(lambda bo, i: (bo + i, 0, 0), b_off // TCB)
        if out is None:
            out = pl.pallas_call(
                _ln_math,
                grid=(nb // TCB,),
                in_specs=[pl.BlockSpec((TCB, seq, HIDDEN // 2),
                                       lambda i: (i, 0, 0))] + common_specs,
                out_specs=pl.BlockSpec((TCB, seq, HIDDEN), om),
                out_shape=out_shape,
            )(gk, tk, pos_table, type_table, gamma, beta)
        else:
            out = pl.pallas_call(
                _tc_ln_chain,
                grid=(nb // TCB,),
                in_specs=[
                    pl.BlockSpec(memory_space=pl.ANY),
                    pl.BlockSpec((TCB, seq, HIDDEN // 2),
                                 lambda i: (i, 0, 0)),
                ] + common_specs,
                out_specs=pl.BlockSpec((TCB, seq, HIDDEN), om),
                out_shape=out_shape,
                input_output_aliases={0: 0},
            )(out, gk, tk, pos_table, type_table, gamma, beta)
        b_off += nb
    return out
